# baseline (speedup 1.0000x reference)
import functools

import jax
import jax.numpy as jnp
from jax import lax
from jax.experimental import pallas as pl
from jax.experimental.pallas import tpu as pltpu
from jax.experimental.pallas import tpu_sc as plsc

EMB_D = 32
NUM_CORES = 2
NUM_SUBCORES = 16
NW = NUM_CORES * NUM_SUBCORES
CHUNK = 128
K = 5
SUP = K * CHUNK


@functools.lru_cache(maxsize=None)
def _make_gather(B: int):
    b_per_w = B // NW
    n_chunks = b_per_w // CHUNK
    n_super = b_per_w // SUP
    assert b_per_w % SUP == 0 and n_super % 2 == 0
    n_iter = n_super // 2
    mesh = plsc.VectorSubcoreMesh(core_axis_name="c", subcore_axis_name="s")

    @functools.partial(
        pl.kernel,
        mesh=mesh,
        out_type=jax.ShapeDtypeStruct((B, EMB_D), jnp.float32),
        scratch_types=[
            pltpu.VMEM((b_per_w,), jnp.int32),
            pltpu.VMEM((SUP, EMB_D), jnp.float32),
            pltpu.VMEM((SUP, EMB_D), jnp.float32),
            pltpu.SemaphoreType.DMA,
            pltpu.SemaphoreType.DMA,
            pltpu.SemaphoreType.DMA,
        ],
        compiler_params=pltpu.CompilerParams(use_tc_tiling_on_sc=False),
    )
    def gather_kernel(idx_hbm, table_hbm, out_hbm, idx_v, rows_a, rows_b,
                      gsem, wsem_a, wsem_b):
        wid = lax.axis_index("s") * NUM_CORES + lax.axis_index("c")
        base = wid * b_per_w

        pltpu.sync_copy(idx_hbm.at[pl.ds(wid * b_per_w, b_per_w)], idx_v)

        def fire_gathers(s, buf):
            for b in range(K):
                pltpu.async_copy(
                    table_hbm.at[idx_v.at[pl.ds((s * K + b) * CHUNK, CHUNK)]],
                    buf.at[pl.ds(b * CHUNK, CHUNK)],
                    gsem,
                )

        def drain_gathers(buf):
            pltpu.make_async_copy(table_hbm.at[pl.ds(0, SUP)], buf, gsem).wait()

        def fire_write(s, buf, wsem):
            pltpu.async_copy(buf, out_hbm.at[pl.ds(base + s * SUP, SUP)], wsem)

        def drain_write(buf, wsem):
            pltpu.make_async_copy(buf, out_hbm.at[pl.ds(0, SUP)], wsem).wait()

        fire_gathers(0, rows_a)

        def body(t, carry):
            s0 = 2 * t
            s1 = s0 + 1
            drain_gathers(rows_a)
            fire_write(s0, rows_a, wsem_a)

            @pl.when(t > 0)
            def _():
                drain_write(rows_b, wsem_b)

            fire_gathers(s1, rows_b)
            drain_gathers(rows_b)
            fire_write(s1, rows_b, wsem_b)
            drain_write(rows_a, wsem_a)

            @pl.when(t < n_iter - 1)
            def _():
                fire_gathers(s0 + 2, rows_a)

            return carry

        lax.fori_loop(0, n_iter, body, 0)
        drain_write(rows_b, wsem_b)

    return gather_kernel


def kernel(x, W):
    rows, cols = x.shape
    B = rows * cols
    xf = x.reshape(B)
    out = _make_gather(B)(xf, W)
    return out.reshape(rows, cols, EMB_D)


# speedup vs baseline: 1.4701x; 1.4701x over previous
import functools

import jax
import jax.numpy as jnp
from jax import lax
from jax.experimental import pallas as pl
from jax.experimental.pallas import tpu as pltpu
from jax.experimental.pallas import tpu_sc as plsc

EMB_D = 32
NUM_CORES = 2
NUM_SUBCORES = 16
NW = NUM_CORES * NUM_SUBCORES
CHUNK = 128
DSTRIDE = 512


@functools.lru_cache(maxsize=None)
def _make_lookup(NT: int, NB_TOTAL: int, V: int):
    NB = NB_TOTAL // NW
    n_g = NB // CHUNK
    assert NT % 2 == 0 and NB % CHUNK == 0
    n_iter = NT // 2
    mesh = plsc.VectorSubcoreMesh(core_axis_name="c", subcore_axis_name="s")

    @functools.partial(
        pl.kernel,
        mesh=mesh,
        out_type=jax.ShapeDtypeStruct((NT, EMB_D, NB_TOTAL), jnp.float32),
        scratch_types=[
            pltpu.VMEM((NB,), jnp.int32),
            pltpu.VMEM((NB,), jnp.int32),
            pltpu.VMEM((NB, EMB_D), jnp.float32),
            pltpu.VMEM((NB, EMB_D), jnp.float32),
            pltpu.VMEM((EMB_D * DSTRIDE,), jnp.float32),
            pltpu.VMEM((EMB_D * DSTRIDE,), jnp.float32),
            pltpu.SemaphoreType.DMA,
            pltpu.SemaphoreType.DMA,
            pltpu.SemaphoreType.DMA,
        ],
        compiler_params=pltpu.CompilerParams(
            use_tc_tiling_on_sc=False, needs_layout_passes=False
        ),
    )
    def lookup_kernel(xt_hbm, table_hbm, out_hbm, idx_a, idx_b, src_a, src_b,
                      dst_a, dst_b, gsem, wsem_a, wsem_b):
        wid = lax.axis_index("s") * NUM_CORES + lax.axis_index("c")
        n0 = wid * NB
        c_lo = lax.iota(jnp.int32, 16) * DSTRIDE
        c_hi = c_lo + 16 * DSTRIDE

        def load_idx(t, ib):
            pltpu.sync_copy(xt_hbm.at[t, pl.ds(n0, NB)], ib)

        def fire_gathers(ib, sb):
            for b in range(n_g):
                pltpu.async_copy(
                    table_hbm.at[ib.at[pl.ds(b * CHUNK, CHUNK)]],
                    sb.at[pl.ds(b * CHUNK, CHUNK)],
                    gsem,
                )

        def drain_gathers(sb):
            pltpu.make_async_copy(table_hbm.at[pl.ds(0, NB)], sb, gsem).wait()

        def transpose(sb, db):
            def row(n, carry):
                v0 = sb[n, pl.ds(0, 16)]
                v1 = sb[n, pl.ds(16, 16)]
                plsc.store_scatter(db, [c_lo + n], v0)
                plsc.store_scatter(db, [c_hi + n], v1)
                return carry

            lax.fori_loop(0, NB, row, 0, unroll=8)

        def fire_writes(t, db, ws):
            for j in range(EMB_D):
                pltpu.async_copy(
                    db.at[pl.ds(j * DSTRIDE, NB)],
                    out_hbm.at[t, j, pl.ds(n0, NB)],
                    ws,
                )

        def drain_writes(sb, ws):
            pltpu.make_async_copy(table_hbm.at[pl.ds(0, NB)], sb, ws).wait()

        load_idx(0, idx_a)
        fire_gathers(idx_a, src_a)

        def body(u, carry):
            t0 = 2 * u
            t1 = t0 + 1
            drain_gathers(src_a)
            load_idx(t1, idx_b)
            fire_gathers(idx_b, src_b)

            @pl.when(u > 0)
            def _():
                drain_writes(src_a, wsem_a)

            transpose(src_a, dst_a)
            fire_writes(t0, dst_a, wsem_a)

            drain_gathers(src_b)

            @pl.when(u < n_iter - 1)
            def _():
                load_idx(t0 + 2, idx_a)
                fire_gathers(idx_a, src_a)

            @pl.when(u > 0)
            def _():
                drain_writes(src_b, wsem_b)

            transpose(src_b, dst_b)
            fire_writes(t1, dst_b, wsem_b)
            return carry

        lax.fori_loop(0, n_iter, body, 0)
        drain_writes(src_a, wsem_a)
        drain_writes(src_b, wsem_b)

    return lookup_kernel


def kernel(x, W):
    rows, cols = x.shape
    xt = x.T
    out3 = _make_lookup(cols, rows, W.shape[0])(xt, W)
    return out3.transpose(2, 0, 1)


# speedup vs baseline: 1.5228x; 1.0359x over previous
import functools

import jax
import jax.numpy as jnp
from jax import lax
from jax.experimental import pallas as pl
from jax.experimental.pallas import tpu as pltpu
from jax.experimental.pallas import tpu_sc as plsc

EMB_D = 32
NUM_CORES = 2
NUM_SUBCORES = 16
NW = NUM_CORES * NUM_SUBCORES
CHUNK = 128
DSTRIDE = 512


@functools.lru_cache(maxsize=None)
def _make_lookup(NT: int, NB_TOTAL: int, V: int):
    NB = NB_TOTAL // NW
    n_g = NB // CHUNK
    assert NT % 2 == 0 and NB % CHUNK == 0
    n_iter = NT // 2
    mesh = plsc.VectorSubcoreMesh(core_axis_name="c", subcore_axis_name="s")

    @functools.partial(
        pl.kernel,
        mesh=mesh,
        out_type=jax.ShapeDtypeStruct((NT, EMB_D, NB_TOTAL), jnp.float32),
        scratch_types=[
            pltpu.VMEM((NB,), jnp.int32),
            pltpu.VMEM((NB,), jnp.int32),
            pltpu.VMEM((NB, EMB_D), jnp.float32),
            pltpu.VMEM((NB, EMB_D), jnp.float32),
            pltpu.VMEM((NB, EMB_D + 1), jnp.float32),
            pltpu.VMEM((EMB_D * DSTRIDE,), jnp.float32),
            pltpu.VMEM((EMB_D * DSTRIDE,), jnp.float32),
            pltpu.SemaphoreType.DMA,
            pltpu.SemaphoreType.DMA,
            pltpu.SemaphoreType.DMA,
        ],
        compiler_params=pltpu.CompilerParams(
            use_tc_tiling_on_sc=False, needs_layout_passes=False
        ),
    )
    def lookup_kernel(xt_hbm, table_hbm, out_hbm, idx_a, idx_b, src_a, src_b,
                      spad, dst_a, dst_b, gsem, wsem_a, wsem_b):
        wid = lax.axis_index("s") * NUM_CORES + lax.axis_index("c")
        n0 = wid * NB
        lane = lax.iota(jnp.int32, 16)

        def load_idx(t, ib):
            pltpu.sync_copy(xt_hbm.at[t, pl.ds(n0, NB)], ib)

        def fire_gathers(ib, sb):
            for b in range(n_g):
                pltpu.async_copy(
                    table_hbm.at[ib.at[pl.ds(b * CHUNK, CHUNK)]],
                    sb.at[pl.ds(b * CHUNK, CHUNK)],
                    gsem,
                )

        def drain_gathers(sb):
            pltpu.make_async_copy(table_hbm.at[pl.ds(0, NB)], sb, gsem).wait()

        def transpose(sb, db):
            def row(n, carry):
                spad[n, pl.ds(0, 16)] = sb[n, pl.ds(0, 16)]
                spad[n, pl.ds(16, 16)] = sb[n, pl.ds(16, 16)]
                return carry

            lax.fori_loop(0, NB, row, 0, unroll=8)

            def blk(nb, carry):
                rows = nb * 16 + lane
                base = nb * 16
                for j in range(EMB_D):
                    v = plsc.load_gather(spad, [rows, lane * 0 + j])
                    db[pl.ds(j * DSTRIDE + base, 16)] = v
                return carry

            lax.fori_loop(0, NB // 16, blk, 0, unroll=2)

        def fire_writes(t, db, ws):
            for j in range(EMB_D):
                pltpu.async_copy(
                    db.at[pl.ds(j * DSTRIDE, NB)],
                    out_hbm.at[t, j, pl.ds(n0, NB)],
                    ws,
                )

        def drain_writes(db, ws):
            pltpu.make_async_copy(
                out_hbm.at[0, 0, pl.ds(0, EMB_D * DSTRIDE)], db, ws
            ).wait()

        load_idx(0, idx_a)
        fire_gathers(idx_a, src_a)

        def body(u, carry):
            t0 = 2 * u
            t1 = t0 + 1
            drain_gathers(src_a)
            load_idx(t1, idx_b)
            fire_gathers(idx_b, src_b)

            @pl.when(u > 0)
            def _():
                drain_writes(dst_a, wsem_a)

            transpose(src_a, dst_a)
            fire_writes(t0, dst_a, wsem_a)

            drain_gathers(src_b)

            @pl.when(u < n_iter - 1)
            def _():
                load_idx(t0 + 2, idx_a)
                fire_gathers(idx_a, src_a)

            @pl.when(u > 0)
            def _():
                drain_writes(dst_b, wsem_b)

            transpose(src_b, dst_b)
            fire_writes(t1, dst_b, wsem_b)
            return carry

        lax.fori_loop(0, n_iter, body, 0)
        drain_writes(dst_a, wsem_a)
        drain_writes(dst_b, wsem_b)

    return lookup_kernel


def kernel(x, W):
    rows, cols = x.shape
    xt = x.T
    out3 = _make_lookup(cols, rows, W.shape[0])(xt, W)
    return out3.transpose(2, 0, 1)


# speedup vs baseline: 1.6717x; 1.0977x over previous
import functools

import jax
import jax.numpy as jnp
from jax import lax
from jax.experimental import pallas as pl
from jax.experimental.pallas import tpu as pltpu
from jax.experimental.pallas import tpu_sc as plsc

EMB_D = 32
NUM_CORES = 2
NUM_SUBCORES = 16
NW = NUM_CORES * NUM_SUBCORES
CHUNK = 128
SPITCH = EMB_D + 1


@functools.lru_cache(maxsize=None)
def _make_lookup(NT: int, NB_TOTAL: int, V: int):
    NB = NB_TOTAL // NW
    n_g = NB // CHUNK
    assert NT % 2 == 0 and NB % CHUNK == 0
    n_iter = NT // 2
    mesh = plsc.VectorSubcoreMesh(core_axis_name="c", subcore_axis_name="s")

    @functools.partial(
        pl.kernel,
        mesh=mesh,
        out_type=jax.ShapeDtypeStruct((NT, EMB_D, NB_TOTAL), jnp.float32),
        scratch_types=[
            pltpu.VMEM((NT, NB), jnp.int32),
            pltpu.VMEM((NB, EMB_D), jnp.float32),
            pltpu.VMEM((NB, EMB_D), jnp.float32),
            pltpu.VMEM((NB * SPITCH,), jnp.float32),
            pltpu.VMEM((EMB_D, NB), jnp.float32),
            pltpu.VMEM((EMB_D, NB), jnp.float32),
            pltpu.SemaphoreType.DMA,
            pltpu.SemaphoreType.DMA,
            pltpu.SemaphoreType.DMA,
        ],
        compiler_params=pltpu.CompilerParams(
            use_tc_tiling_on_sc=False, needs_layout_passes=False
        ),
    )
    def lookup_kernel(xt_hbm, table_hbm, out_hbm, idx_v, src_a, src_b,
                      spad, dst_a, dst_b, gsem, wsem_a, wsem_b):
        wid = lax.axis_index("s") * NUM_CORES + lax.axis_index("c")
        n0 = wid * NB
        lane = lax.iota(jnp.int32, 16)

        pltpu.sync_copy(xt_hbm.at[pl.ds(0, NT), pl.ds(n0, NB)], idx_v)

        def fire_gathers(t, sb):
            for b in range(n_g):
                pltpu.async_copy(
                    table_hbm.at[idx_v.at[t, pl.ds(b * CHUNK, CHUNK)]],
                    sb.at[pl.ds(b * CHUNK, CHUNK)],
                    gsem,
                )

        def drain_gathers(sb):
            pltpu.make_async_copy(table_hbm.at[pl.ds(0, NB)], sb, gsem).wait()

        def transpose(sb, db):
            def row(n, carry):
                p = n * SPITCH
                spad[pl.ds(p, 16)] = sb[n, pl.ds(0, 16)]
                spad[pl.ds(p + 16, 16)] = sb[n, pl.ds(16, 16)]
                return carry

            lax.fori_loop(0, NB, row, 0, unroll=8)

            def blk(nb, carry):
                flat = (nb * 16 + lane) * SPITCH
                base = nb * 16
                for j in range(EMB_D):
                    v = plsc.load_gather(spad, [flat + j])
                    db[j, pl.ds(base, 16)] = v
                return carry

            lax.fori_loop(0, NB // 16, blk, 0, unroll=2)

        def fire_write(t, db, ws):
            pltpu.async_copy(
                db,
                out_hbm.at[t, pl.ds(0, EMB_D), pl.ds(n0, NB)],
                ws,
            )

        def drain_write(db, ws):
            pltpu.make_async_copy(
                out_hbm.at[0, pl.ds(0, EMB_D), pl.ds(0, NB)], db, ws
            ).wait()

        fire_gathers(0, src_a)

        def body(u, carry):
            t0 = 2 * u
            t1 = t0 + 1
            drain_gathers(src_a)
            fire_gathers(t1, src_b)

            @pl.when(u > 0)
            def _():
                drain_write(dst_a, wsem_a)

            transpose(src_a, dst_a)
            fire_write(t0, dst_a, wsem_a)

            drain_gathers(src_b)

            @pl.when(u < n_iter - 1)
            def _():
                fire_gathers(t0 + 2, src_a)

            @pl.when(u > 0)
            def _():
                drain_write(dst_b, wsem_b)

            transpose(src_b, dst_b)
            fire_write(t1, dst_b, wsem_b)
            return carry

        lax.fori_loop(0, n_iter, body, 0)
        drain_write(dst_a, wsem_a)
        drain_write(dst_b, wsem_b)

    return lookup_kernel


def kernel(x, W):
    rows, cols = x.shape
    xt = x.T
    out3 = _make_lookup(cols, rows, W.shape[0])(xt, W)
    return out3.transpose(2, 0, 1)
